# Initial kernel scaffold; baseline (speedup 1.0000x reference)
import jax
import jax.numpy as jnp
from jax.experimental import pallas as pl


def kernel(x, edge_index, W, b):
    raise NotImplementedError("write your pallas kernel here")


# speedup vs baseline: 3.9653x; 3.9653x over previous
import functools

import jax
import jax.numpy as jnp
from jax import lax
from jax.experimental import pallas as pl
from jax.experimental.pallas import tpu as pltpu
from jax.experimental.pallas import tpu_sc as plsc

N_NODES = 10000
N_EDGES = 320000
D = 128

NC = 2
NS = 16
NW = NC * NS

CHUNK = 128
STEPS = -(-N_EDGES // (NW * CHUNK))
E_PAD = NW * CHUNK * STEPS
N_PAD = 10240
ROWS_PER_TILE = N_PAD // NS


def _sc_segment_sum(x, src, dst):
    mesh = plsc.VectorSubcoreMesh(core_axis_name="c", subcore_axis_name="s")

    @functools.partial(
        pl.kernel,
        out_type=jax.ShapeDtypeStruct((NC, N_PAD, D), jnp.float32),
        mesh=mesh,
        scratch_types=[
            pltpu.VMEM((CHUNK,), jnp.int32),
            pltpu.VMEM((CHUNK,), jnp.int32),
            pltpu.VMEM((CHUNK, D), jnp.float32),
            pltpu.VMEM_SHARED((N_PAD, D), jnp.float32),
            pltpu.SemaphoreType.DMA,
        ],
    )
    def run(x_hbm, src_hbm, dst_hbm, out_hbm, sidx, didx, rows, hacc, sem):
        c = lax.axis_index("c")
        s = lax.axis_index("s")
        wid = s * NC + c

        zeros16 = jnp.zeros((16,), jnp.float32)

        def zero_row(i, _):
            for j in range(D // 16):
                rows[i, pl.ds(j * 16, 16)] = zeros16
            return 0

        lax.fori_loop(0, CHUNK, zero_row, 0)

        def zero_acc(k, _):
            pltpu.sync_copy(rows, hacc.at[pl.ds(s * ROWS_PER_TILE + k * CHUNK, CHUNK)])
            return 0

        lax.fori_loop(0, ROWS_PER_TILE // CHUNK, zero_acc, 0)
        plsc.subcore_barrier()

        base_w = wid * (STEPS * CHUNK)

        def step(g, _):
            off = base_w + g * CHUNK
            pltpu.sync_copy(src_hbm.at[pl.ds(off, CHUNK)], sidx)
            pltpu.sync_copy(dst_hbm.at[pl.ds(off, CHUNK)], didx)
            pltpu.async_copy(x_hbm.at[sidx], rows, sem).wait()
            pltpu.sync_copy(rows, hacc.at[didx], add=True)
            return 0

        lax.fori_loop(0, STEPS, step, 0)
        plsc.subcore_barrier()

        r0 = s * ROWS_PER_TILE
        pltpu.sync_copy(hacc.at[pl.ds(r0, ROWS_PER_TILE)],
                        out_hbm.at[c, pl.ds(r0, ROWS_PER_TILE)])

    return run(x, src, dst)


def _tc_linear_body(h0_ref, h1_ref, wt_ref, b_ref, o_ref):
    h = h0_ref[...] + h1_ref[...]
    o_ref[...] = jnp.dot(h, wt_ref[...], preferred_element_type=jnp.float32) + b_ref[...]


def _tc_linear(h0, h1, wt, b):
    bm = 512
    return pl.pallas_call(
        _tc_linear_body,
        grid=(N_PAD // bm,),
        in_specs=[
            pl.BlockSpec((bm, D), lambda i: (i, 0)),
            pl.BlockSpec((bm, D), lambda i: (i, 0)),
            pl.BlockSpec((D, D), lambda i: (0, 0)),
            pl.BlockSpec((1, D), lambda i: (0, 0)),
        ],
        out_specs=pl.BlockSpec((bm, D), lambda i: (i, 0)),
        out_shape=jax.ShapeDtypeStruct((N_PAD, D), jnp.float32),
    )(h0, h1, wt, b)


def kernel(x, edge_index, W, b):
    ei = edge_index.astype(jnp.int32)
    pad = E_PAD - N_EDGES
    src = jnp.concatenate([ei[0], jnp.zeros((pad,), jnp.int32)])
    dst = jnp.concatenate([ei[1], jnp.full((pad,), N_NODES, jnp.int32)])

    partials = _sc_segment_sum(x, src, dst)
    out = _tc_linear(partials[0], partials[1], W.T, b.reshape(1, D))
    return out[:N_NODES]
